# baseline (speedup 1.0000x reference)
import functools

import jax
import jax.numpy as jnp
from jax import lax
from jax.experimental import pallas as pl
from jax.experimental.pallas import tpu as pltpu
from jax.experimental.pallas import tpu_sc as plsc

D_MODEL = 768
LANES = 16
VPR = D_MODEL // LANES
NC = 2
NS = 16
NW = NC * NS
CHUNK = 32


@functools.partial(jax.jit, static_argnums=(3, 4))
def _embed(x, token_emb, pos_emb, batch, seq_len):
    s_per_w = seq_len // NW
    halves = s_per_w // CHUNK
    n_chunks = batch * halves
    mesh = plsc.VectorSubcoreMesh(core_axis_name="c", subcore_axis_name="s")

    @functools.partial(
        pl.kernel,
        out_type=jax.ShapeDtypeStruct((batch, seq_len, D_MODEL), jnp.float32),
        mesh=mesh,
        scratch_types=[
            pltpu.VMEM((batch, s_per_w), jnp.int32),
            pltpu.VMEM((s_per_w, D_MODEL), jnp.float32),
            pltpu.VMEM((CHUNK, D_MODEL), jnp.float32),
            pltpu.VMEM((CHUNK, D_MODEL), jnp.float32),
            pltpu.SemaphoreType.DMA,
            pltpu.SemaphoreType.DMA,
            pltpu.SemaphoreType.DMA,
            pltpu.SemaphoreType.DMA,
        ],
    )
    def body(x_hbm, tok_hbm, pos_hbm, out_hbm,
             idx_v, pos_buf, buf0, buf1, g0, g1, s0, s1):
        wid = lax.axis_index("s") * NC + lax.axis_index("c")
        s_base = wid * s_per_w
        bufs = (buf0, buf1)
        gsems = (g0, g1)
        ssems = (s0, s1)

        pltpu.sync_copy(pos_hbm.at[pl.ds(s_base, s_per_w)], pos_buf)
        for b in range(batch):
            pltpu.sync_copy(x_hbm.at[b, pl.ds(s_base, s_per_w)], idx_v.at[b])

        def start_gather(i):
            b, h = divmod(i, halves)
            j = i % 2
            return pltpu.async_copy(
                tok_hbm.at[idx_v.at[b, pl.ds(h * CHUNK, CHUNK)]],
                bufs[j], gsems[j],
            )

        def start_store(i):
            b, h = divmod(i, halves)
            j = i % 2
            return pltpu.async_copy(
                bufs[j],
                out_hbm.at[b, pl.ds(s_base + h * CHUNK, CHUNK)],
                ssems[j],
            )

        gathers = [None, None]
        stores = [None, None]
        gathers[0] = start_gather(0)
        for i in range(n_chunks):
            j = i % 2
            nj = (i + 1) % 2
            if i + 1 < n_chunks:
                if stores[nj] is not None:
                    stores[nj].wait()
                gathers[nj] = start_gather(i + 1)
            gathers[j].wait()

            h = i % halves
            buf = bufs[j]

            def add_row(r, _):
                for k in range(VPR):
                    sl = pl.ds(k * LANES, LANES)
                    plsc.addupdate(buf.at[r, sl], pos_buf[h * CHUNK + r, sl])
                return ()

            lax.fori_loop(0, CHUNK, add_row, (), unroll=False)
            stores[j] = start_store(i)
        stores[0].wait()
        stores[1].wait()

    return body(x, token_emb, pos_emb)


def kernel(x, token_emb, pos_emb):
    batch, seq = x.shape
    return _embed(x.astype(jnp.int32), token_emb, pos_emb, batch, seq)


# speedup vs baseline: 1.2713x; 1.2713x over previous
import functools

import jax
import jax.numpy as jnp
from jax import lax
from jax.experimental import pallas as pl
from jax.experimental.pallas import tpu as pltpu
from jax.experimental.pallas import tpu_sc as plsc

D_MODEL = 768
LANES = 16
VPR = D_MODEL // LANES
NC = 2
NS = 16
NW = NC * NS
POS_PER_CHUNK = 8
REG_BLOCK = 16


@functools.partial(jax.jit, static_argnums=(3, 4))
def _embed(xr, token_emb, pos_emb, batch, seq_len):
    s_per_w = seq_len // NW
    n_chunks = s_per_w // POS_PER_CHUNK
    chunk_rows = POS_PER_CHUNK * batch
    mesh = plsc.VectorSubcoreMesh(core_axis_name="c", subcore_axis_name="s")

    @functools.partial(
        pl.kernel,
        out_type=jax.ShapeDtypeStruct((batch, seq_len, D_MODEL), jnp.float32),
        mesh=mesh,
        scratch_types=[
            pltpu.VMEM((n_chunks, chunk_rows), jnp.int32),
            pltpu.VMEM((s_per_w, D_MODEL), jnp.float32),
            pltpu.VMEM((chunk_rows, D_MODEL), jnp.float32),
            pltpu.VMEM((chunk_rows, D_MODEL), jnp.float32),
            pltpu.SemaphoreType.DMA,
            pltpu.SemaphoreType.DMA,
            pltpu.SemaphoreType.DMA,
            pltpu.SemaphoreType.DMA,
        ],
    )
    def body(xr_hbm, tok_hbm, pos_hbm, out_hbm,
             idx_v, pos_buf, buf0, buf1, g0, g1, s0, s1):
        wid = lax.axis_index("s") * NC + lax.axis_index("c")
        s_base = wid * s_per_w
        bufs = (buf0, buf1)
        gsems = (g0, g1)
        ssems = (s0, s1)

        pltpu.sync_copy(pos_hbm.at[pl.ds(s_base, s_per_w)], pos_buf)
        pltpu.sync_copy(xr_hbm.at[wid], idx_v)

        def start_gather(c):
            j = c % 2
            return pltpu.async_copy(
                tok_hbm.at[idx_v.at[c]], bufs[j], gsems[j])

        def start_stores(c):
            j = c % 2
            return [
                pltpu.async_copy(
                    bufs[j].at[pl.ds(b * POS_PER_CHUNK, POS_PER_CHUNK)],
                    out_hbm.at[b, pl.ds(s_base + c * POS_PER_CHUNK,
                                        POS_PER_CHUNK)],
                    ssems[j],
                )
                for b in range(batch)
            ]

        gathers = [None, None]
        stores = [None, None]
        gathers[0] = start_gather(0)
        for c in range(n_chunks):
            j = c % 2
            nj = (c + 1) % 2
            if c + 1 < n_chunks:
                if stores[nj] is not None:
                    for hnd in stores[nj]:
                        hnd.wait()
                gathers[nj] = start_gather(c + 1)
            gathers[j].wait()

            buf = bufs[j]

            def add_pos(p, _):
                row = c * POS_PER_CHUNK + p
                for t in range(VPR // REG_BLOCK):
                    regs = [
                        pos_buf[row, pl.ds((t * REG_BLOCK + v) * LANES, LANES)]
                        for v in range(REG_BLOCK)
                    ]
                    for b in range(batch):
                        r = b * POS_PER_CHUNK + p
                        for v in range(REG_BLOCK):
                            sl = pl.ds((t * REG_BLOCK + v) * LANES, LANES)
                            plsc.addupdate(buf.at[r, sl], regs[v])
                return ()

            lax.fori_loop(0, POS_PER_CHUNK, add_pos, (), unroll=False)
            stores[j] = start_stores(c)
        for hnd in stores[0]:
            hnd.wait()
        for hnd in stores[1]:
            hnd.wait()

    return body(xr, token_emb, pos_emb)


def kernel(x, token_emb, pos_emb):
    batch, seq = x.shape
    s_per_w = seq // NW
    n_chunks = s_per_w // POS_PER_CHUNK
    xr = (x.astype(jnp.int32)
          .reshape(batch, NW, n_chunks, POS_PER_CHUNK)
          .transpose(1, 2, 0, 3)
          .reshape(NW, n_chunks, batch * POS_PER_CHUNK))
    return _embed(xr, token_emb, pos_emb, batch, seq)


# speedup vs baseline: 1.3391x; 1.0533x over previous
import functools

import jax
import jax.numpy as jnp
from jax import lax
from jax.experimental import pallas as pl
from jax.experimental.pallas import tpu as pltpu
from jax.experimental.pallas import tpu_sc as plsc

D_MODEL = 768
LANES = 16
VPR = D_MODEL // LANES
NC = 2
NS = 16
NW = NC * NS
POS_PER_CHUNK = 8
REG_BLOCK = 16
NRING = 3


@functools.partial(jax.jit, static_argnums=(3, 4))
def _embed(xr, token_emb, pos_emb, batch, seq_len):
    s_per_w = seq_len // NW
    n_chunks = s_per_w // POS_PER_CHUNK
    chunk_rows = POS_PER_CHUNK * batch
    mesh = plsc.VectorSubcoreMesh(core_axis_name="c", subcore_axis_name="s")

    @functools.partial(
        pl.kernel,
        out_type=jax.ShapeDtypeStruct((batch, seq_len, D_MODEL), jnp.float32),
        mesh=mesh,
        scratch_types=[
            pltpu.VMEM((n_chunks, chunk_rows), jnp.int32),
            pltpu.VMEM((s_per_w, D_MODEL), jnp.float32),
            pltpu.VMEM((chunk_rows, D_MODEL), jnp.float32),
            pltpu.VMEM((chunk_rows, D_MODEL), jnp.float32),
            pltpu.VMEM((chunk_rows, D_MODEL), jnp.float32),
            pltpu.SemaphoreType.DMA,
            pltpu.SemaphoreType.DMA,
            pltpu.SemaphoreType.DMA,
            pltpu.SemaphoreType.DMA,
            pltpu.SemaphoreType.DMA,
            pltpu.SemaphoreType.DMA,
            pltpu.SemaphoreType.DMA,
        ],
    )
    def body(xr_hbm, tok_hbm, pos_hbm, out_hbm,
             idx_v, pos_buf, buf0, buf1, buf2,
             g0, g1, g2, s0, s1, s2, psem):
        wid = lax.axis_index("s") * NC + lax.axis_index("c")
        s_base = wid * s_per_w
        bufs = (buf0, buf1, buf2)
        gsems = (g0, g1, g2)
        ssems = (s0, s1, s2)

        pltpu.sync_copy(xr_hbm.at[wid], idx_v)
        pos_cp = pltpu.async_copy(
            pos_hbm.at[pl.ds(s_base, s_per_w)], pos_buf, psem)

        def start_gather(c):
            j = c % NRING
            return pltpu.async_copy(
                tok_hbm.at[idx_v.at[c]], bufs[j], gsems[j])

        def start_stores(c):
            j = c % NRING
            return [
                pltpu.async_copy(
                    bufs[j].at[pl.ds(b * POS_PER_CHUNK, POS_PER_CHUNK)],
                    out_hbm.at[b, pl.ds(s_base + c * POS_PER_CHUNK,
                                        POS_PER_CHUNK)],
                    ssems[j],
                )
                for b in range(batch)
            ]

        gathers = [None] * NRING
        stores = [None] * NRING
        gathers[0] = start_gather(0)
        gathers[1] = start_gather(1)
        pos_cp.wait()
        for c in range(n_chunks):
            j = c % NRING
            k = (c + 2) % NRING
            if c + 2 < n_chunks:
                if stores[k] is not None:
                    for hnd in stores[k]:
                        hnd.wait()
                gathers[k] = start_gather(c + 2)
            gathers[j].wait()

            buf = bufs[j]

            def add_pos(p, _):
                row = c * POS_PER_CHUNK + p
                for t in range(VPR // REG_BLOCK):
                    regs = [
                        pos_buf[row, pl.ds((t * REG_BLOCK + v) * LANES, LANES)]
                        for v in range(REG_BLOCK)
                    ]
                    for b in range(batch):
                        r = b * POS_PER_CHUNK + p
                        for v in range(REG_BLOCK):
                            sl = pl.ds((t * REG_BLOCK + v) * LANES, LANES)
                            plsc.addupdate(buf.at[r, sl], regs[v])
                return ()

            lax.fori_loop(0, POS_PER_CHUNK, add_pos, (), unroll=False)
            stores[j] = start_stores(c)
        for sset in stores:
            if sset is not None:
                for hnd in sset:
                    hnd.wait()

    return body(xr, token_emb, pos_emb)


def kernel(x, token_emb, pos_emb):
    batch, seq = x.shape
    s_per_w = seq // NW
    n_chunks = s_per_w // POS_PER_CHUNK
    xr = (x.astype(jnp.int32)
          .reshape(batch, NW, n_chunks, POS_PER_CHUNK)
          .transpose(1, 2, 0, 3)
          .reshape(NW, n_chunks, batch * POS_PER_CHUNK))
    return _embed(xr, token_emb, pos_emb, batch, seq)
